# baseline (speedup 1.0000x reference)
import jax
import jax.numpy as jnp
from jax.experimental import pallas as pl
from jax.experimental.pallas import tpu as pltpu

_H = 512
_W = 512
_BAND = 16
_CB = 512
_K = 16
_T = 21.0
_DUP_B = 3


def _raster_kernel(starts_ref, ncks_ref, params_ref, out_ref, alpha_s):
    b = pl.program_id(0)
    cb = pl.program_id(1)
    t = b * (_W // _CB) + cb
    start = starts_ref[t]
    nck = ncks_ref[t]
    xs = (jax.lax.broadcasted_iota(jnp.int32, (1, _CB), 1)
          + cb * _CB).astype(jnp.float32) + 0.5
    y0 = (b * _BAND).astype(jnp.float32)
    out_ref[...] = jnp.zeros_like(out_ref)

    def body(i, carry):
        off = start + i * _K
        p = params_ref[pl.ds(off, _K), :]
        cx = p[:, 0:1]
        cy = p[:, 1:2]
        c0 = p[:, 2:3]
        c1 = p[:, 3:4]
        c2 = p[:, 4:5]
        w = p[:, 5:8]
        dx = xs - cx
        a = (0.5 * c0) * dx * dx
        c1dx = c1 * dx
        for y in range(_BAND):
            dy = (y0 + (y + 0.5)) - cy
            sig = a + (0.5 * c2) * (dy * dy) + dy * c1dx
            alpha_s[:, pl.ds(y * _CB, _CB)] = jnp.exp(-sig)
        out_ref[...] += jax.lax.dot_general(
            w, alpha_s[...], (((0,), (0,)), ((), ())),
            preferred_element_type=jnp.float32)
        return carry

    jax.lax.fori_loop(0, nck, body, 0)


def kernel(embed):
    e = embed.reshape(-1, 9).astype(jnp.float32)
    n = e.shape[0]
    xy = jnp.tanh(e[:, :2])
    cx = 0.5 * _W * (xy[:, 0] + 1.0)
    cy = 0.5 * _H * (xy[:, 1] + 1.0)
    l0 = e[:, 5] + 0.5
    l1 = e[:, 6]
    l2 = e[:, 7] + 0.5
    cov00 = l0 * l0
    cov01 = l0 * l1
    cov11 = l1 * l1 + l2 * l2
    det = cov00 * cov11 - cov01 * cov01
    conic0 = cov11 / det
    conic1 = -cov01 / det
    conic2 = cov00 / det
    w = e[:, 2:5] * jax.nn.sigmoid(e[:, 8:9])

    half_tr = 0.5 * (cov00 + cov11)
    lmax = half_tr + jnp.sqrt((0.5 * (cov00 - cov11)) ** 2 + cov01 * cov01)
    r = jnp.sqrt(2.0 * _T * lmax)

    P = jnp.concatenate(
        [jnp.stack([cx, cy, conic0, conic1, conic2], axis=1), w], axis=1)

    nb = _H // _BAND
    nc = _W // _CB
    nt = nb * nc
    dup_c = 1 if nc == 1 else 2

    blo = jnp.ceil((cy - r - (_BAND - 0.5)) / _BAND).astype(jnp.int32)
    bhi = jnp.floor((cy + r - 0.5) / _BAND).astype(jnp.int32)
    bb = blo[:, None] + jnp.arange(_DUP_B, dtype=jnp.int32)[None, :]
    bvalid = (bb <= bhi[:, None]) & (bb >= 0) & (bb < nb)
    if nc == 1:
        cc = jnp.zeros((n, 1), jnp.int32)
        cvalid = jnp.ones((n, 1), bool)
    else:
        clo = jnp.ceil((cx - r - (_CB - 0.5)) / _CB).astype(jnp.int32)
        chi = jnp.floor((cx + r - 0.5) / _CB).astype(jnp.int32)
        cc = clo[:, None] + jnp.arange(dup_c, dtype=jnp.int32)[None, :]
        cvalid = (cc <= chi[:, None]) & (cc >= 0) & (cc < nc)
    tid = (bb[:, :, None] * nc + cc[:, None, :]).reshape(-1)
    valid = (bvalid[:, :, None] & cvalid[:, None, :]).reshape(-1)
    ne = n * _DUP_B * dup_c
    gid = jnp.broadcast_to(
        jnp.arange(n, dtype=jnp.int32)[:, None, None],
        (n, _DUP_B, dup_c)).reshape(-1)

    onehot = ((tid[:, None] == jnp.arange(nt, dtype=jnp.int32)[None, :])
              & valid[:, None]).astype(jnp.float32)
    blk = 512
    nblk = ne // blk
    oh3 = onehot.reshape(nblk, blk, nt)
    tril = jnp.tril(jnp.ones((blk, blk), jnp.float32))
    within = jnp.einsum('ij,cjt->cit', tril, oh3,
                        preferred_element_type=jnp.float32)
    bsum = oh3.sum(axis=1)
    bpre = jnp.cumsum(bsum, axis=0) - bsum
    incl = (within + bpre[:, None, :]).reshape(ne, nt)
    rank = jnp.sum(incl * onehot, axis=1) - 1.0
    counts = bsum.sum(axis=0)
    ncks = jnp.ceil(counts / _K).astype(jnp.int32)
    poff = _K * jnp.concatenate(
        [jnp.zeros((1,), jnp.int32), jnp.cumsum(ncks)])[:nt]
    nslot = ne + nt * _K
    tclip = jnp.clip(tid, 0, nt - 1)
    pos = poff[tclip] + rank.astype(jnp.int32)
    pos = jnp.where(valid, pos, nslot)
    src = jnp.full((nslot,), n, jnp.int32).at[pos].set(gid, mode='drop')
    P_ext = jnp.concatenate([P, jnp.zeros((1, 8), jnp.float32)], axis=0)
    E2 = P_ext[src]

    grid_spec = pltpu.PrefetchScalarGridSpec(
        num_scalar_prefetch=2,
        grid=(nb, nc),
        in_specs=[pl.BlockSpec((nslot, 8), lambda b, c, *_: (0, 0))],
        out_specs=pl.BlockSpec((3, _BAND * _CB), lambda b, c, *_: (0, b * nc + c)),
        scratch_shapes=[pltpu.VMEM((_K, _BAND * _CB), jnp.float32)],
    )
    out = pl.pallas_call(
        _raster_kernel,
        grid_spec=grid_spec,
        out_shape=jax.ShapeDtypeStruct((3, _H * _W), jnp.float32),
    )(poff, ncks, E2)
    return out.reshape(3, _H, _W)[None]


# speedup vs baseline: 1.0696x; 1.0696x over previous
import jax
import jax.numpy as jnp
from jax.experimental import pallas as pl
from jax.experimental.pallas import tpu as pltpu

_H = 512
_W = 512
_BAND = 16
_CB = 512
_K = 32
_T = 18.0
_DUP_B = 3


def _raster_kernel(starts_ref, ncks_ref, params_ref, out_ref, alpha_s):
    b = pl.program_id(0)
    cb = pl.program_id(1)
    t = b * (_W // _CB) + cb
    start = starts_ref[t]
    nck = ncks_ref[t]
    xs = (jax.lax.broadcasted_iota(jnp.int32, (1, _CB), 1)
          + cb * _CB).astype(jnp.float32) + 0.5
    y0 = (b * _BAND).astype(jnp.float32)
    out_ref[...] = jnp.zeros_like(out_ref)

    def body(i, carry):
        off = start + i * _K
        p = params_ref[pl.ds(off, _K), :]
        cx = p[:, 0:1]
        cy = p[:, 1:2]
        c0 = p[:, 2:3]
        c1 = p[:, 3:4]
        c2 = p[:, 4:5]
        w = p[:, 5:8]
        dx = xs - cx
        a = (0.5 * c0) * dx * dx
        c1dx = c1 * dx
        for y in range(_BAND):
            dy = (y0 + (y + 0.5)) - cy
            sig = a + (0.5 * c2) * (dy * dy) + dy * c1dx
            alpha_s[:, pl.ds(y * _CB, _CB)] = jnp.exp(-sig)
        out_ref[...] += jax.lax.dot_general(
            w, alpha_s[...], (((0,), (0,)), ((), ())),
            preferred_element_type=jnp.float32)
        return carry

    jax.lax.fori_loop(0, nck, body, 0)


def kernel(embed):
    e = embed.reshape(-1, 9).astype(jnp.float32)
    n = e.shape[0]
    xy = jnp.tanh(e[:, :2])
    cx = 0.5 * _W * (xy[:, 0] + 1.0)
    cy = 0.5 * _H * (xy[:, 1] + 1.0)
    l0 = e[:, 5] + 0.5
    l1 = e[:, 6]
    l2 = e[:, 7] + 0.5
    cov00 = l0 * l0
    cov01 = l0 * l1
    cov11 = l1 * l1 + l2 * l2
    det = cov00 * cov11 - cov01 * cov01
    conic0 = cov11 / det
    conic1 = -cov01 / det
    conic2 = cov00 / det
    w = e[:, 2:5] * jax.nn.sigmoid(e[:, 8:9])

    half_tr = 0.5 * (cov00 + cov11)
    lmax = half_tr + jnp.sqrt((0.5 * (cov00 - cov11)) ** 2 + cov01 * cov01)
    r = jnp.sqrt(2.0 * _T * lmax)

    P = jnp.concatenate(
        [jnp.stack([cx, cy, conic0, conic1, conic2], axis=1), w], axis=1)

    nb = _H // _BAND
    nc = _W // _CB
    nt = nb * nc
    dup_c = 1 if nc == 1 else 2

    blo = jnp.ceil((cy - r - (_BAND - 0.5)) / _BAND).astype(jnp.int32)
    bhi = jnp.floor((cy + r - 0.5) / _BAND).astype(jnp.int32)
    bb = blo[:, None] + jnp.arange(_DUP_B, dtype=jnp.int32)[None, :]
    bvalid = (bb <= bhi[:, None]) & (bb >= 0) & (bb < nb)
    if nc == 1:
        cc = jnp.zeros((n, 1), jnp.int32)
        cvalid = jnp.ones((n, 1), bool)
    else:
        clo = jnp.ceil((cx - r - (_CB - 0.5)) / _CB).astype(jnp.int32)
        chi = jnp.floor((cx + r - 0.5) / _CB).astype(jnp.int32)
        cc = clo[:, None] + jnp.arange(dup_c, dtype=jnp.int32)[None, :]
        cvalid = (cc <= chi[:, None]) & (cc >= 0) & (cc < nc)
    tid = (bb[:, :, None] * nc + cc[:, None, :]).reshape(-1)
    valid = (bvalid[:, :, None] & cvalid[:, None, :]).reshape(-1)
    ne = n * _DUP_B * dup_c
    gid = jnp.broadcast_to(
        jnp.arange(n, dtype=jnp.int32)[:, None, None],
        (n, _DUP_B, dup_c)).reshape(-1)

    onehot = ((tid[:, None] == jnp.arange(nt, dtype=jnp.int32)[None, :])
              & valid[:, None]).astype(jnp.float32)
    blk = 512
    nblk = ne // blk
    oh3 = onehot.reshape(nblk, blk, nt)
    tril = jnp.tril(jnp.ones((blk, blk), jnp.float32))
    within = jnp.einsum('ij,cjt->cit', tril, oh3,
                        preferred_element_type=jnp.float32)
    bsum = oh3.sum(axis=1)
    bpre = jnp.cumsum(bsum, axis=0) - bsum
    incl = (within + bpre[:, None, :]).reshape(ne, nt)
    rank = jnp.sum(incl * onehot, axis=1) - 1.0
    counts = bsum.sum(axis=0)
    ncks = jnp.ceil(counts / _K).astype(jnp.int32)
    poff = _K * jnp.concatenate(
        [jnp.zeros((1,), jnp.int32), jnp.cumsum(ncks)])[:nt]
    nslot = ne + nt * _K
    tclip = jnp.clip(tid, 0, nt - 1)
    pos = poff[tclip] + rank.astype(jnp.int32)
    pos = jnp.where(valid, pos, nslot)
    src = jnp.full((nslot,), n, jnp.int32).at[pos].set(gid, mode='drop')
    P_ext = jnp.concatenate([P, jnp.zeros((1, 8), jnp.float32)], axis=0)
    E2 = P_ext[src]

    grid_spec = pltpu.PrefetchScalarGridSpec(
        num_scalar_prefetch=2,
        grid=(nb, nc),
        in_specs=[pl.BlockSpec((nslot, 8), lambda b, c, *_: (0, 0))],
        out_specs=pl.BlockSpec((3, _BAND * _CB), lambda b, c, *_: (0, b * nc + c)),
        scratch_shapes=[pltpu.VMEM((_K, _BAND * _CB), jnp.float32)],
    )
    out = pl.pallas_call(
        _raster_kernel,
        grid_spec=grid_spec,
        out_shape=jax.ShapeDtypeStruct((3, _H * _W), jnp.float32),
    )(poff, ncks, E2)
    return out.reshape(3, _H, _W)[None]
